# baseline (speedup 1.0000x reference)
import functools

import jax
import jax.numpy as jnp
from jax import lax
from jax.experimental import pallas as pl
from jax.experimental.pallas import tpu as pltpu

_BT = 256
_NBUF = 4


def _body(actual_ref, x_ref, o_ref, buf, acc, sems):
    B, T, D = x_ref.shape

    nbs, cums = [], [jnp.int32(0)]
    for j in range(B):
        nb = jnp.maximum((actual_ref[j] + _BT - 1) // _BT, 1)
        nbs.append(nb)
        cums.append(cums[-1] + nb)
    total = cums[-1]

    def locate(g):
        b = jnp.int32(0)
        for j in range(1, B):
            b = b + (g >= cums[j]).astype(jnp.int32)
        cum_b = jnp.int32(0)
        nb_b = jnp.int32(0)
        for j in range(B):
            is_j = (b == j).astype(jnp.int32)
            cum_b = cum_b + is_j * cums[j]
            nb_b = nb_b + is_j * nbs[j]
        t0 = (g - cum_b) * _BT
        return b, t0, cum_b, nb_b

    def copy_args(g, slot):
        b, t0, _, _ = locate(g)
        return (x_ref.at[b, pl.ds(t0, _BT), :], buf.at[slot], sems.at[slot])

    def issue(g, slot):
        pltpu.make_async_copy(*copy_args(g, slot)).start()

    for k in range(_NBUF):
        @pl.when(k < total)
        def _prime(k=k):
            issue(jnp.int32(k), k)

    def chunk_step(g, carry):
        slot = lax.rem(g, _NBUF)
        pltpu.make_async_copy(*copy_args(g, slot)).wait()
        b, t0, cum_b, nb_b = locate(g)
        n = actual_ref[b]

        row = lax.broadcasted_iota(jnp.int32, (1, _BT), 1) + t0
        w = (row < n).astype(jnp.float32)
        partial = lax.dot_general(
            w, buf[slot], (((1,), (0,)), ((), ())),
            preferred_element_type=jnp.float32)
        acc[...] = jnp.where(t0 == 0, partial, acc[...] + partial)

        @pl.when(g + _NBUF < total)
        def _next():
            issue(g + _NBUF, slot)

        @pl.when(g + 1 == cum_b + nb_b)
        def _flush():
            o_ref[pl.ds(b, 1), 0, :] = acc[...] / n.astype(jnp.float32)

        return carry

    lax.fori_loop(0, total, chunk_step, 0)


def kernel(x, lens):
    B, T, D = x.shape
    actual = jnp.floor(lens * T).astype(jnp.int32)

    grid_spec = pltpu.PrefetchScalarGridSpec(
        num_scalar_prefetch=1,
        grid=(1,),
        in_specs=[pl.BlockSpec(memory_space=pl.ANY)],
        out_specs=pl.BlockSpec((B, 1, D), lambda i, actual_ref: (0, 0, 0)),
        scratch_shapes=[
            pltpu.VMEM((_NBUF, _BT, D), jnp.float32),
            pltpu.VMEM((1, D), jnp.float32),
            pltpu.SemaphoreType.DMA((_NBUF,)),
        ],
    )
    out = pl.pallas_call(
        _body,
        grid_spec=grid_spec,
        out_shape=jax.ShapeDtypeStruct((B, 1, D), jnp.float32),
    )(actual, x)
    return out.reshape(B, D)
